# baseline (speedup 1.0000x reference)
import functools

import jax
import jax.numpy as jnp
from jax import lax
from jax.experimental import pallas as pl
from jax.experimental.pallas import tpu as pltpu
from jax.experimental.pallas import tpu_sc as plsc

N = 10000
NPAD = 10240
E = 320000
CH = 128
ER = 2560
EPAD = ER * CH
CHL = 64
ERL = EPAD // CHL
NCHT = ERL // 16
NCHW = ERL // 32
ROWS = 2 * NPAD
ZR = ROWS // 16
ZR2 = NPAD // 16

_mesh = plsc.VectorSubcoreMesh(core_axis_name="c", subcore_axis_name="s")
_sc_params = pltpu.CompilerParams(use_tc_tiling_on_sc=False)


@functools.partial(
    pl.kernel,
    out_type=jax.ShapeDtypeStruct((2, ROWS, 64), jnp.float32),
    mesh=_mesh,
    compiler_params=_sc_params,
    scratch_types=[
        pltpu.VMEM((NCHT, CHL), jnp.int32),
        pltpu.VMEM((NCHT, CHL), jnp.int32),
        pltpu.VMEM((CHL, 64), jnp.float32),
        pltpu.VMEM((CHL, 64), jnp.float32),
        pltpu.VMEM_SHARED((ROWS, 64), jnp.float32),
        pltpu.SemaphoreType.DMA,
        pltpu.SemaphoreType.DMA,
        pltpu.SemaphoreType.DMA,
        pltpu.SemaphoreType.DMA,
    ],
)
def _sc_layer_agg(g_hbm, gidx_hbm, sidx_hbm, zeros_hbm, out_hbm,
                  gidx_v, sidx_v, r0, r1, acc_sh,
                  gs0, gs1, ss0, ss1):
    rows = (r0, r1)
    gsems = (gs0, gs1)
    ssems = (ss0, ss1)
    c = lax.axis_index("c")
    s = lax.axis_index("s")
    base = s * NCHT
    pltpu.sync_copy(gidx_hbm.at[pl.ds(base, NCHT)], gidx_v)
    pltpu.sync_copy(sidx_hbm.at[pl.ds(base, NCHT)], sidx_v)
    pltpu.sync_copy(zeros_hbm.at[pl.ds(s * ZR, ZR)], acc_sh.at[pl.ds(s * ZR, ZR)])
    plsc.subcore_barrier()

    pltpu.async_copy(g_hbm.at[c].at[gidx_v.at[0]], rows[0], gsems[0])
    pltpu.make_async_copy(g_hbm.at[c].at[gidx_v.at[0]], rows[0], gsems[0]).wait()
    pltpu.async_copy(rows[0], acc_sh.at[sidx_v.at[0]], ssems[0], add=True)
    pltpu.async_copy(g_hbm.at[c].at[gidx_v.at[1]], rows[1], gsems[1])

    def ring(i, carry):
        j0 = 2 * i + 1
        for b in (1, 0):
            j = j0 + (1 - b)
            pltpu.make_async_copy(
                g_hbm.at[c].at[gidx_v.at[j]], rows[b], gsems[b]).wait()
            pltpu.make_async_copy(
                rows[1 - b], acc_sh.at[sidx_v.at[j - 1]], ssems[1 - b]).wait()
            jn = jnp.minimum(j + 1, NCHT - 1)
            pltpu.async_copy(g_hbm.at[c].at[gidx_v.at[jn]], rows[1 - b], gsems[1 - b])
            pltpu.async_copy(rows[b], acc_sh.at[sidx_v.at[j]], ssems[b], add=True)
        return carry

    lax.fori_loop(0, (NCHT - 2) // 2, ring, 0)
    pltpu.make_async_copy(
        g_hbm.at[c].at[gidx_v.at[NCHT - 1]], rows[1], gsems[1]).wait()
    pltpu.async_copy(rows[1], acc_sh.at[sidx_v.at[NCHT - 1]], ssems[1], add=True)
    pltpu.make_async_copy(
        rows[0], acc_sh.at[sidx_v.at[NCHT - 2]], ssems[0]).wait()
    pltpu.make_async_copy(
        rows[1], acc_sh.at[sidx_v.at[NCHT - 1]], ssems[1]).wait()
    plsc.subcore_barrier()
    pltpu.sync_copy(acc_sh.at[pl.ds(s * ZR, ZR)], out_hbm.at[c].at[pl.ds(s * ZR, ZR)])


@functools.partial(
    pl.kernel,
    out_type=jax.ShapeDtypeStruct((2, NPAD, 16), jnp.float32),
    mesh=_mesh,
    compiler_params=_sc_params,
    scratch_types=[
        pltpu.VMEM((NCHW, CHL), jnp.int32),
        pltpu.VMEM((NCHW, CHL), jnp.int32),
        pltpu.VMEM((CHL, 16), jnp.float32),
        pltpu.VMEM((CHL, 16), jnp.float32),
        pltpu.VMEM_SHARED((NPAD, 16), jnp.float32),
        pltpu.SemaphoreType.DMA,
        pltpu.SemaphoreType.DMA,
        pltpu.SemaphoreType.DMA,
        pltpu.SemaphoreType.DMA,
    ],
)
def _sc_last_agg(g_hbm, gidx_hbm, sidx_hbm, zeros_hbm, out_hbm,
                 gidx_v, sidx_v, r0, r1, acc_sh,
                 gs0, gs1, ss0, ss1):
    rows = (r0, r1)
    gsems = (gs0, gs1)
    ssems = (ss0, ss1)
    c = lax.axis_index("c")
    s = lax.axis_index("s")
    base = (c * 16 + s) * NCHW
    pltpu.sync_copy(gidx_hbm.at[pl.ds(base, NCHW)], gidx_v)
    pltpu.sync_copy(sidx_hbm.at[pl.ds(base, NCHW)], sidx_v)
    pltpu.sync_copy(zeros_hbm.at[pl.ds(s * ZR2, ZR2)], acc_sh.at[pl.ds(s * ZR2, ZR2)])
    plsc.subcore_barrier()

    pltpu.async_copy(g_hbm.at[gidx_v.at[0]], rows[0], gsems[0])
    pltpu.make_async_copy(g_hbm.at[gidx_v.at[0]], rows[0], gsems[0]).wait()
    pltpu.async_copy(rows[0], acc_sh.at[sidx_v.at[0]], ssems[0], add=True)
    pltpu.async_copy(g_hbm.at[gidx_v.at[1]], rows[1], gsems[1])

    def ring(i, carry):
        j0 = 2 * i + 1
        for b in (1, 0):
            j = j0 + (1 - b)
            pltpu.make_async_copy(
                g_hbm.at[gidx_v.at[j]], rows[b], gsems[b]).wait()
            pltpu.make_async_copy(
                rows[1 - b], acc_sh.at[sidx_v.at[j - 1]], ssems[1 - b]).wait()
            jn = jnp.minimum(j + 1, NCHW - 1)
            pltpu.async_copy(g_hbm.at[gidx_v.at[jn]], rows[1 - b], gsems[1 - b])
            pltpu.async_copy(rows[b], acc_sh.at[sidx_v.at[j]], ssems[b], add=True)
        return carry

    lax.fori_loop(0, (NCHW - 2) // 2, ring, 0)
    pltpu.make_async_copy(g_hbm.at[gidx_v.at[NCHW - 1]], rows[1], gsems[1]).wait()
    pltpu.async_copy(rows[1], acc_sh.at[sidx_v.at[NCHW - 1]], ssems[1], add=True)
    pltpu.make_async_copy(
        rows[0], acc_sh.at[sidx_v.at[NCHW - 2]], ssems[0]).wait()
    pltpu.make_async_copy(
        rows[1], acc_sh.at[sidx_v.at[NCHW - 1]], ssems[1]).wait()
    plsc.subcore_barrier()
    pltpu.sync_copy(acc_sh.at[pl.ds(s * ZR2, ZR2)], out_hbm.at[c].at[pl.ds(s * ZR2, ZR2)])


def _prep_idx_body(src_ref, dst_ref, rev_ref, sidx_ref, gidx_ref):
    rev = rev_ref[...]
    sidx_ref[...] = dst_ref[...] + NPAD * rev
    gidx_ref[...] = src_ref[...] + NPAD * rev


def _tc_prep_idx(srcp, dstp, revp):
    blk = pl.BlockSpec((32, CH), lambda r: (r, 0))
    return pl.pallas_call(
        _prep_idx_body,
        grid=(ER // 32,),
        in_specs=[blk, blk, blk],
        out_specs=[blk, blk],
        out_shape=[jax.ShapeDtypeStruct((ER, CH), jnp.int32)] * 2,
    )(srcp, dstp, revp)


def _deg_fin_body(dac_ref, dis_ref, disall_ref):
    dac = dac_ref[...]
    cnt_st = dac[0, 0, :, 0:1]
    cnt_ts = dac[0, 1, :, 0:1]
    dis_ref[0] = lax.rsqrt(cnt_st + 1.0)
    dis_ref[1] = lax.rsqrt(cnt_ts + 1.0)
    disall_ref[...] = lax.rsqrt(cnt_st + cnt_ts + 1.0)


def _tc_deg_fin(degacc4):
    return pl.pallas_call(
        _deg_fin_body,
        grid=(NPAD // 256,),
        in_specs=[pl.BlockSpec((2, 2, 256, 64), lambda r: (0, 0, r, 0))],
        out_specs=[pl.BlockSpec((2, 256, 1), lambda r: (0, r, 0)),
                   pl.BlockSpec((256, 1), lambda r: (r, 0))],
        out_shape=[jax.ShapeDtypeStruct((2, NPAD, 1), jnp.float32),
                   jax.ShapeDtypeStruct((NPAD, 1), jnp.float32)],
    )(degacc4)


def _fwd_body(h_ref, w_ref, dis_ref, out_ref):
    hw = jnp.dot(h_ref[...], w_ref[0], preferred_element_type=jnp.float32)
    out_ref[0, 0] = dis_ref[0] * hw


def _tc_fwd(h, wcat, dis):
    fin = h.shape[1]
    wq = jnp.transpose(wcat.reshape(fin, 4, 64), (1, 0, 2))
    return pl.pallas_call(
        _fwd_body,
        grid=(NPAD // 256, 2, 2),
        in_specs=[pl.BlockSpec((256, fin), lambda r, p, c: (r, 0)),
                  pl.BlockSpec((1, fin, 64), lambda r, p, c: (2 * p + c, 0, 0)),
                  pl.BlockSpec((1, 256, 1), lambda r, p, c: (p, r, 0))],
        out_specs=pl.BlockSpec((1, 1, 256, 64), lambda r, p, c: (c, p, r, 0)),
        out_shape=jax.ShapeDtypeStruct((2, 2, NPAD, 64), jnp.float32),
    )(h, wq, dis)


def _epi_body(acc_ref, g_ref, dis_ref, bq_ref, h_ref):
    a = acc_ref[...]
    g = g_ref[...]
    d = dis_ref[...]
    cols = []
    for p in range(2):
        for cc in range(2):
            cols.append(jnp.maximum(d[p] * (a[cc, p] + g[cc, p]) + bq_ref[cc, p], 0.0))
    h_ref[...] = jnp.concatenate(cols, axis=1)


def _tc_epi(acc4, g4, dis, bq):
    return pl.pallas_call(
        _epi_body,
        grid=(NPAD // 256,),
        in_specs=[pl.BlockSpec((2, 2, 256, 64), lambda r: (0, 0, r, 0)),
                  pl.BlockSpec((2, 2, 256, 64), lambda r: (0, 0, r, 0)),
                  pl.BlockSpec((2, 256, 1), lambda r: (0, r, 0)),
                  pl.BlockSpec((2, 2, 1, 64), lambda r: (0, 0, 0, 0))],
        out_specs=pl.BlockSpec((256, 256), lambda r: (r, 0)),
        out_shape=jax.ShapeDtypeStruct((NPAD, 256), jnp.float32),
    )(acc4, g4, dis, bq)


def _lastmm_body(h_ref, w_ref, dis_ref, g_ref):
    hw = jnp.dot(h_ref[...], w_ref[...], preferred_element_type=jnp.float32)
    g_ref[...] = dis_ref[...] * hw


def _tc_lastmm(h, w_last, dis_all):
    return pl.pallas_call(
        _lastmm_body,
        grid=(NPAD // 256,),
        in_specs=[pl.BlockSpec((256, 256), lambda r: (r, 0)),
                  pl.BlockSpec((256, 16), lambda r: (0, 0)),
                  pl.BlockSpec((256, 1), lambda r: (r, 0))],
        out_specs=pl.BlockSpec((256, 16), lambda r: (r, 0)),
        out_shape=jax.ShapeDtypeStruct((NPAD, 16), jnp.float32),
    )(h, w_last, dis_all)


def _final_body(acc_ref, g_ref, dis_ref, b_ref, out_ref):
    o = dis_ref[...] * (acc_ref[0] + acc_ref[1] + g_ref[...]) + b_ref[...]
    m = jnp.max(o, axis=1, keepdims=True)
    e = o - m
    out_ref[...] = e - jnp.log(jnp.sum(jnp.exp(e), axis=1, keepdims=True))


def _tc_final(acc2, g2, dis_all, b_last):
    return pl.pallas_call(
        _final_body,
        grid=(NPAD // 256,),
        in_specs=[pl.BlockSpec((2, 256, 16), lambda r: (0, r, 0)),
                  pl.BlockSpec((256, 16), lambda r: (r, 0)),
                  pl.BlockSpec((256, 1), lambda r: (r, 0)),
                  pl.BlockSpec((1, 16), lambda r: (0, 0))],
        out_specs=pl.BlockSpec((256, 16), lambda r: (r, 0)),
        out_shape=jax.ShapeDtypeStruct((NPAD, 16), jnp.float32),
    )(acc2, g2, dis_all, b_last)


def kernel(x, edge_index, is_reversed, W_st0, b_st0, W_ts0, b_ts0,
           W_st1, b_st1, W_ts1, b_ts1, W_last, b_last):
    src = edge_index[0]
    dst = edge_index[1]
    rev = is_reversed.astype(jnp.int32)
    pad = EPAD - E
    srcp = jnp.pad(src, (0, pad)).reshape(ER, CH)
    dstp = jnp.pad(dst, (0, pad), constant_values=N).reshape(ER, CH)
    revp = jnp.pad(rev, (0, pad)).reshape(ER, CH)

    sidx, gidx = _tc_prep_idx(srcp, dstp, revp)
    sidx = sidx.reshape(ERL, CHL)
    gidx = gidx.reshape(ERL, CHL)

    zeros16 = jnp.zeros((ROWS, 16), jnp.float32)
    zeros64 = jnp.zeros((ROWS, 64), jnp.float32)

    ones_tab = jnp.zeros((2, ROWS, 64), jnp.float32).at[:, 0, :].set(1.0)
    gidx0 = jnp.zeros((ERL, CHL), jnp.int32)
    degacc = _sc_layer_agg(ones_tab, gidx0, sidx, zeros64)
    dis, dis_all = _tc_deg_fin(degacc.reshape(2, 2, NPAD, 64))

    xpad = jnp.pad(x, ((0, NPAD - N), (0, 0)))
    wcat0 = jnp.concatenate([W_st0, W_ts0], axis=1)
    wcat1 = jnp.concatenate([W_st1, W_ts1], axis=1)

    def bias_quads(b_st, b_ts):
        return jnp.stack([
            jnp.stack([b_st[0:64], b_ts[0:64]]),
            jnp.stack([b_st[64:128], b_ts[64:128]]),
        ])[:, :, None, :]

    bq0 = bias_quads(b_st0, b_ts0)
    bq1 = bias_quads(b_st1, b_ts1)

    g0 = _tc_fwd(xpad, wcat0, dis)
    acc0 = _sc_layer_agg(g0.reshape(2, ROWS, 64), gidx, sidx, zeros64)
    h1 = _tc_epi(acc0.reshape(2, 2, NPAD, 64), g0, dis, bq0)

    g1 = _tc_fwd(h1, wcat1, dis)
    acc1 = _sc_layer_agg(g1.reshape(2, ROWS, 64), gidx, sidx, zeros64)
    h2 = _tc_epi(acc1.reshape(2, 2, NPAD, 64), g1, dis, bq1)

    g2 = _tc_lastmm(h2, W_last, dis_all)
    acc2 = _sc_last_agg(g2, srcp.reshape(ERL, CHL), dstp.reshape(ERL, CHL), zeros16)
    out = _tc_final(acc2, g2, dis_all, b_last[None, :])
    return out[:N]


# speedup vs baseline: 4.7248x; 4.7248x over previous
import functools

import jax
import jax.numpy as jnp
from jax import lax
from jax.experimental import pallas as pl
from jax.experimental.pallas import tpu as pltpu
from jax.experimental.pallas import tpu_sc as plsc

N = 10000
NPAD = 10240
E = 320000
CH = 128
ER = 2560
EPAD = ER * CH
CHL = 64
ERL = EPAD // CHL
NCHT = ERL // 16
NCHW = ERL // 32
ROWS = 2 * NPAD
ZR = ROWS // 16
ZR2 = NPAD // 16

_mesh = plsc.VectorSubcoreMesh(core_axis_name="c", subcore_axis_name="s")
_sc_params = pltpu.CompilerParams(use_tc_tiling_on_sc=False)


@functools.partial(
    pl.kernel,
    out_type=jax.ShapeDtypeStruct((2, ROWS, 16), jnp.float32),
    mesh=_mesh,
    compiler_params=_sc_params,
    scratch_types=[
        pltpu.VMEM((NCHW + 2, CHL), jnp.int32),
        pltpu.VMEM((CHL, 16), jnp.float32),
        pltpu.VMEM_SHARED((ROWS, 16), jnp.float32),
        pltpu.SemaphoreType.DMA,
        pltpu.SemaphoreType.DMA,
    ],
)
def _sc_degree(sidx_hbm, ones_hbm, zeros_hbm, out_hbm, sidx_v, ones_v, acc_sh,
               ss0, ss1):
    ssems = (ss0, ss1)
    c = lax.axis_index("c")
    s = lax.axis_index("s")
    base = (c * 16 + s) * NCHW
    pltpu.sync_copy(sidx_hbm.at[pl.ds(base, NCHW)], sidx_v.at[pl.ds(0, NCHW)])
    trash = jnp.full((16,), N, jnp.int32)
    for k in range(CHL // 16):
        sidx_v[NCHW, pl.ds(k * 16, 16)] = trash
        sidx_v[NCHW + 1, pl.ds(k * 16, 16)] = trash
    pltpu.sync_copy(ones_hbm, ones_v)
    pltpu.sync_copy(zeros_hbm.at[pl.ds(s * ZR, ZR)], acc_sh.at[pl.ds(s * ZR, ZR)])
    plsc.subcore_barrier()

    pltpu.async_copy(ones_v, acc_sh.at[sidx_v.at[0]], ssems[0], add=True)
    pltpu.async_copy(ones_v, acc_sh.at[sidx_v.at[1]], ssems[1], add=True)

    def body(i, carry):
        j = 2 * i
        for b in range(2):
            pltpu.make_async_copy(
                ones_v, acc_sh.at[sidx_v.at[j + b]], ssems[b]).wait()
            pltpu.async_copy(
                ones_v, acc_sh.at[sidx_v.at[j + b + 2]], ssems[b], add=True)
        return carry

    lax.fori_loop(0, NCHW // 2, body, 0)
    for b in range(2):
        pltpu.make_async_copy(
            ones_v, acc_sh.at[sidx_v.at[NCHW + b]], ssems[b]).wait()
    plsc.subcore_barrier()
    pltpu.sync_copy(acc_sh.at[pl.ds(s * ZR, ZR)], out_hbm.at[c].at[pl.ds(s * ZR, ZR)])


@functools.partial(
    pl.kernel,
    out_type=jax.ShapeDtypeStruct((2, ROWS, 64), jnp.float32),
    mesh=_mesh,
    compiler_params=_sc_params,
    scratch_types=[
        pltpu.VMEM((NCHT, CHL), jnp.int32),
        pltpu.VMEM((NCHT, CHL), jnp.int32),
        pltpu.VMEM((CHL, 64), jnp.float32),
        pltpu.VMEM((CHL, 64), jnp.float32),
        pltpu.VMEM_SHARED((ROWS, 64), jnp.float32),
        pltpu.SemaphoreType.DMA,
        pltpu.SemaphoreType.DMA,
        pltpu.SemaphoreType.DMA,
        pltpu.SemaphoreType.DMA,
    ],
)
def _sc_layer_agg(g_hbm, gidx_hbm, sidx_hbm, zeros_hbm, out_hbm,
                  gidx_v, sidx_v, r0, r1, acc_sh,
                  gs0, gs1, ss0, ss1):
    rows = (r0, r1)
    gsems = (gs0, gs1)
    ssems = (ss0, ss1)
    c = lax.axis_index("c")
    s = lax.axis_index("s")
    base = s * NCHT
    pltpu.sync_copy(gidx_hbm.at[pl.ds(base, NCHT)], gidx_v)
    pltpu.sync_copy(sidx_hbm.at[pl.ds(base, NCHT)], sidx_v)
    pltpu.sync_copy(zeros_hbm.at[pl.ds(s * ZR, ZR)], acc_sh.at[pl.ds(s * ZR, ZR)])
    plsc.subcore_barrier()

    pltpu.async_copy(g_hbm.at[c].at[gidx_v.at[0]], rows[0], gsems[0])
    pltpu.make_async_copy(g_hbm.at[c].at[gidx_v.at[0]], rows[0], gsems[0]).wait()
    pltpu.async_copy(rows[0], acc_sh.at[sidx_v.at[0]], ssems[0], add=True)
    pltpu.async_copy(g_hbm.at[c].at[gidx_v.at[1]], rows[1], gsems[1])

    def ring(i, carry):
        j0 = 2 * i + 1
        for b in (1, 0):
            j = j0 + (1 - b)
            pltpu.make_async_copy(
                g_hbm.at[c].at[gidx_v.at[j]], rows[b], gsems[b]).wait()
            pltpu.make_async_copy(
                rows[1 - b], acc_sh.at[sidx_v.at[j - 1]], ssems[1 - b]).wait()
            jn = jnp.minimum(j + 1, NCHT - 1)
            pltpu.async_copy(g_hbm.at[c].at[gidx_v.at[jn]], rows[1 - b], gsems[1 - b])
            pltpu.async_copy(rows[b], acc_sh.at[sidx_v.at[j]], ssems[b], add=True)
        return carry

    lax.fori_loop(0, (NCHT - 2) // 2, ring, 0)
    pltpu.make_async_copy(
        g_hbm.at[c].at[gidx_v.at[NCHT - 1]], rows[1], gsems[1]).wait()
    pltpu.async_copy(rows[1], acc_sh.at[sidx_v.at[NCHT - 1]], ssems[1], add=True)
    pltpu.make_async_copy(
        rows[0], acc_sh.at[sidx_v.at[NCHT - 2]], ssems[0]).wait()
    pltpu.make_async_copy(
        rows[1], acc_sh.at[sidx_v.at[NCHT - 1]], ssems[1]).wait()
    plsc.subcore_barrier()
    pltpu.sync_copy(acc_sh.at[pl.ds(s * ZR, ZR)], out_hbm.at[c].at[pl.ds(s * ZR, ZR)])


@functools.partial(
    pl.kernel,
    out_type=jax.ShapeDtypeStruct((2, NPAD, 16), jnp.float32),
    mesh=_mesh,
    compiler_params=_sc_params,
    scratch_types=[
        pltpu.VMEM((NCHW, CHL), jnp.int32),
        pltpu.VMEM((NCHW, CHL), jnp.int32),
        pltpu.VMEM((CHL, 16), jnp.float32),
        pltpu.VMEM((CHL, 16), jnp.float32),
        pltpu.VMEM_SHARED((NPAD, 16), jnp.float32),
        pltpu.SemaphoreType.DMA,
        pltpu.SemaphoreType.DMA,
        pltpu.SemaphoreType.DMA,
        pltpu.SemaphoreType.DMA,
    ],
)
def _sc_last_agg(g_hbm, gidx_hbm, sidx_hbm, zeros_hbm, out_hbm,
                 gidx_v, sidx_v, r0, r1, acc_sh,
                 gs0, gs1, ss0, ss1):
    rows = (r0, r1)
    gsems = (gs0, gs1)
    ssems = (ss0, ss1)
    c = lax.axis_index("c")
    s = lax.axis_index("s")
    base = (c * 16 + s) * NCHW
    pltpu.sync_copy(gidx_hbm.at[pl.ds(base, NCHW)], gidx_v)
    pltpu.sync_copy(sidx_hbm.at[pl.ds(base, NCHW)], sidx_v)
    pltpu.sync_copy(zeros_hbm.at[pl.ds(s * ZR2, ZR2)], acc_sh.at[pl.ds(s * ZR2, ZR2)])
    plsc.subcore_barrier()

    pltpu.async_copy(g_hbm.at[gidx_v.at[0]], rows[0], gsems[0])
    pltpu.make_async_copy(g_hbm.at[gidx_v.at[0]], rows[0], gsems[0]).wait()
    pltpu.async_copy(rows[0], acc_sh.at[sidx_v.at[0]], ssems[0], add=True)
    pltpu.async_copy(g_hbm.at[gidx_v.at[1]], rows[1], gsems[1])

    def ring(i, carry):
        j0 = 2 * i + 1
        for b in (1, 0):
            j = j0 + (1 - b)
            pltpu.make_async_copy(
                g_hbm.at[gidx_v.at[j]], rows[b], gsems[b]).wait()
            pltpu.make_async_copy(
                rows[1 - b], acc_sh.at[sidx_v.at[j - 1]], ssems[1 - b]).wait()
            jn = jnp.minimum(j + 1, NCHW - 1)
            pltpu.async_copy(g_hbm.at[gidx_v.at[jn]], rows[1 - b], gsems[1 - b])
            pltpu.async_copy(rows[b], acc_sh.at[sidx_v.at[j]], ssems[b], add=True)
        return carry

    lax.fori_loop(0, (NCHW - 2) // 2, ring, 0)
    pltpu.make_async_copy(g_hbm.at[gidx_v.at[NCHW - 1]], rows[1], gsems[1]).wait()
    pltpu.async_copy(rows[1], acc_sh.at[sidx_v.at[NCHW - 1]], ssems[1], add=True)
    pltpu.make_async_copy(
        rows[0], acc_sh.at[sidx_v.at[NCHW - 2]], ssems[0]).wait()
    pltpu.make_async_copy(
        rows[1], acc_sh.at[sidx_v.at[NCHW - 1]], ssems[1]).wait()
    plsc.subcore_barrier()
    pltpu.sync_copy(acc_sh.at[pl.ds(s * ZR2, ZR2)], out_hbm.at[c].at[pl.ds(s * ZR2, ZR2)])


def _prep_idx_body(src_ref, dst_ref, rev_ref, sidx_ref, gidx_ref):
    rev = rev_ref[...]
    sidx_ref[...] = dst_ref[...] + NPAD * rev
    gidx_ref[...] = src_ref[...] + NPAD * rev


def _tc_prep_idx(srcp, dstp, revp):
    blk = pl.BlockSpec((32, CH), lambda r: (r, 0))
    return pl.pallas_call(
        _prep_idx_body,
        grid=(ER // 32,),
        in_specs=[blk, blk, blk],
        out_specs=[blk, blk],
        out_shape=[jax.ShapeDtypeStruct((ER, CH), jnp.int32)] * 2,
    )(srcp, dstp, revp)


def _deg_fin_body(dac_ref, dis_ref, disall_ref):
    dac = dac_ref[...]
    cnt_st = dac[0, 0, :, 0:1] + dac[1, 0, :, 0:1]
    cnt_ts = dac[0, 1, :, 0:1] + dac[1, 1, :, 0:1]
    dis_ref[0] = lax.rsqrt(cnt_st + 1.0)
    dis_ref[1] = lax.rsqrt(cnt_ts + 1.0)
    disall_ref[...] = lax.rsqrt(cnt_st + cnt_ts + 1.0)


def _tc_deg_fin(degacc4):
    return pl.pallas_call(
        _deg_fin_body,
        grid=(NPAD // 256,),
        in_specs=[pl.BlockSpec((2, 2, 256, 16), lambda r: (0, 0, r, 0))],
        out_specs=[pl.BlockSpec((2, 256, 1), lambda r: (0, r, 0)),
                   pl.BlockSpec((256, 1), lambda r: (r, 0))],
        out_shape=[jax.ShapeDtypeStruct((2, NPAD, 1), jnp.float32),
                   jax.ShapeDtypeStruct((NPAD, 1), jnp.float32)],
    )(degacc4)


def _fwd_body(h_ref, w_ref, dis_ref, out_ref):
    hw = jnp.dot(h_ref[...], w_ref[0], preferred_element_type=jnp.float32)
    out_ref[0, 0] = dis_ref[0] * hw


def _tc_fwd(h, wcat, dis):
    fin = h.shape[1]
    wq = jnp.transpose(wcat.reshape(fin, 4, 64), (1, 0, 2))
    return pl.pallas_call(
        _fwd_body,
        grid=(NPAD // 256, 2, 2),
        in_specs=[pl.BlockSpec((256, fin), lambda r, p, c: (r, 0)),
                  pl.BlockSpec((1, fin, 64), lambda r, p, c: (2 * p + c, 0, 0)),
                  pl.BlockSpec((1, 256, 1), lambda r, p, c: (p, r, 0))],
        out_specs=pl.BlockSpec((1, 1, 256, 64), lambda r, p, c: (c, p, r, 0)),
        out_shape=jax.ShapeDtypeStruct((2, 2, NPAD, 64), jnp.float32),
    )(h, wq, dis)


def _epi_body(acc_ref, g_ref, dis_ref, bq_ref, h_ref):
    a = acc_ref[...]
    g = g_ref[...]
    d = dis_ref[...]
    cols = []
    for p in range(2):
        for cc in range(2):
            cols.append(jnp.maximum(d[p] * (a[cc, p] + g[cc, p]) + bq_ref[cc, p], 0.0))
    h_ref[...] = jnp.concatenate(cols, axis=1)


def _tc_epi(acc4, g4, dis, bq):
    return pl.pallas_call(
        _epi_body,
        grid=(NPAD // 256,),
        in_specs=[pl.BlockSpec((2, 2, 256, 64), lambda r: (0, 0, r, 0)),
                  pl.BlockSpec((2, 2, 256, 64), lambda r: (0, 0, r, 0)),
                  pl.BlockSpec((2, 256, 1), lambda r: (0, r, 0)),
                  pl.BlockSpec((2, 2, 1, 64), lambda r: (0, 0, 0, 0))],
        out_specs=pl.BlockSpec((256, 256), lambda r: (r, 0)),
        out_shape=jax.ShapeDtypeStruct((NPAD, 256), jnp.float32),
    )(acc4, g4, dis, bq)


def _lastmm_body(h_ref, w_ref, dis_ref, g_ref):
    hw = jnp.dot(h_ref[...], w_ref[...], preferred_element_type=jnp.float32)
    g_ref[...] = dis_ref[...] * hw


def _tc_lastmm(h, w_last, dis_all):
    return pl.pallas_call(
        _lastmm_body,
        grid=(NPAD // 256,),
        in_specs=[pl.BlockSpec((256, 256), lambda r: (r, 0)),
                  pl.BlockSpec((256, 16), lambda r: (0, 0)),
                  pl.BlockSpec((256, 1), lambda r: (r, 0))],
        out_specs=pl.BlockSpec((256, 16), lambda r: (r, 0)),
        out_shape=jax.ShapeDtypeStruct((NPAD, 16), jnp.float32),
    )(h, w_last, dis_all)


def _final_body(acc_ref, g_ref, dis_ref, b_ref, out_ref):
    o = dis_ref[...] * (acc_ref[0] + acc_ref[1] + g_ref[...]) + b_ref[...]
    m = jnp.max(o, axis=1, keepdims=True)
    e = o - m
    out_ref[...] = e - jnp.log(jnp.sum(jnp.exp(e), axis=1, keepdims=True))


def _tc_final(acc2, g2, dis_all, b_last):
    return pl.pallas_call(
        _final_body,
        grid=(NPAD // 256,),
        in_specs=[pl.BlockSpec((2, 256, 16), lambda r: (0, r, 0)),
                  pl.BlockSpec((256, 16), lambda r: (r, 0)),
                  pl.BlockSpec((256, 1), lambda r: (r, 0)),
                  pl.BlockSpec((1, 16), lambda r: (0, 0))],
        out_specs=pl.BlockSpec((256, 16), lambda r: (r, 0)),
        out_shape=jax.ShapeDtypeStruct((NPAD, 16), jnp.float32),
    )(acc2, g2, dis_all, b_last)


def kernel(x, edge_index, is_reversed, W_st0, b_st0, W_ts0, b_ts0,
           W_st1, b_st1, W_ts1, b_ts1, W_last, b_last):
    src = edge_index[0]
    dst = edge_index[1]
    rev = is_reversed.astype(jnp.int32)
    pad = EPAD - E
    srcp = jnp.pad(src, (0, pad)).reshape(ER, CH)
    dstp = jnp.pad(dst, (0, pad), constant_values=N).reshape(ER, CH)
    revp = jnp.pad(rev, (0, pad)).reshape(ER, CH)

    sidx, gidx = _tc_prep_idx(srcp, dstp, revp)
    sidx = sidx.reshape(ERL, CHL)
    gidx = gidx.reshape(ERL, CHL)

    zeros16 = jnp.zeros((ROWS, 16), jnp.float32)
    zeros64 = jnp.zeros((ROWS, 64), jnp.float32)
    ones16 = jnp.ones((CHL, 16), jnp.float32)

    degacc = _sc_degree(sidx, ones16, zeros16)
    dis, dis_all = _tc_deg_fin(degacc.reshape(2, 2, NPAD, 16))

    xpad = jnp.pad(x, ((0, NPAD - N), (0, 0)))
    wcat0 = jnp.concatenate([W_st0, W_ts0], axis=1)
    wcat1 = jnp.concatenate([W_st1, W_ts1], axis=1)

    def bias_quads(b_st, b_ts):
        return jnp.stack([
            jnp.stack([b_st[0:64], b_ts[0:64]]),
            jnp.stack([b_st[64:128], b_ts[64:128]]),
        ])[:, :, None, :]

    bq0 = bias_quads(b_st0, b_ts0)
    bq1 = bias_quads(b_st1, b_ts1)

    g0 = _tc_fwd(xpad, wcat0, dis)
    acc0 = _sc_layer_agg(g0.reshape(2, ROWS, 64), gidx, sidx, zeros64)
    h1 = _tc_epi(acc0.reshape(2, 2, NPAD, 64), g0, dis, bq0)

    g1 = _tc_fwd(h1, wcat1, dis)
    acc1 = _sc_layer_agg(g1.reshape(2, ROWS, 64), gidx, sidx, zeros64)
    h2 = _tc_epi(acc1.reshape(2, 2, NPAD, 64), g1, dis, bq1)

    g2 = _tc_lastmm(h2, W_last, dis_all)
    acc2 = _sc_last_agg(g2, srcp.reshape(ERL, CHL), dstp.reshape(ERL, CHL), zeros16)
    out = _tc_final(acc2, g2, dis_all, b_last[None, :])
    return out[:N]


# speedup vs baseline: 5.1828x; 1.0969x over previous
import functools

import jax
import jax.numpy as jnp
from jax import lax
from jax.experimental import pallas as pl
from jax.experimental.pallas import tpu as pltpu
from jax.experimental.pallas import tpu_sc as plsc

N = 10000
NPAD = 10240
E = 320000
CH = 128
ER = 2560
EPAD = ER * CH
CHL = 64
ERL = EPAD // CHL
NCHT = ER // 16
NCHW = ERL // 32
ROWS = 2 * NPAD
ZR = ROWS // 16
ZR2 = NPAD // 16

_mesh = plsc.VectorSubcoreMesh(core_axis_name="c", subcore_axis_name="s")
_sc_params = pltpu.CompilerParams(use_tc_tiling_on_sc=False)


@functools.partial(
    pl.kernel,
    out_type=jax.ShapeDtypeStruct((2, ROWS, 16), jnp.float32),
    mesh=_mesh,
    compiler_params=_sc_params,
    scratch_types=[
        pltpu.VMEM((NCHW + 2, CHL), jnp.int32),
        pltpu.VMEM((CHL, 16), jnp.float32),
        pltpu.VMEM_SHARED((ROWS, 16), jnp.float32),
        pltpu.SemaphoreType.DMA,
        pltpu.SemaphoreType.DMA,
    ],
)
def _sc_degree(sidx_hbm, ones_hbm, zeros_hbm, out_hbm, sidx_v, ones_v, acc_sh,
               ss0, ss1):
    ssems = (ss0, ss1)
    c = lax.axis_index("c")
    s = lax.axis_index("s")
    base = (c * 16 + s) * NCHW
    pltpu.sync_copy(sidx_hbm.at[pl.ds(base, NCHW)], sidx_v.at[pl.ds(0, NCHW)])
    trash = jnp.full((16,), N, jnp.int32)
    for k in range(CHL // 16):
        sidx_v[NCHW, pl.ds(k * 16, 16)] = trash
        sidx_v[NCHW + 1, pl.ds(k * 16, 16)] = trash
    pltpu.sync_copy(ones_hbm, ones_v)
    pltpu.sync_copy(zeros_hbm.at[pl.ds(s * ZR, ZR)], acc_sh.at[pl.ds(s * ZR, ZR)])
    plsc.subcore_barrier()

    pltpu.async_copy(ones_v, acc_sh.at[sidx_v.at[0]], ssems[0], add=True)
    pltpu.async_copy(ones_v, acc_sh.at[sidx_v.at[1]], ssems[1], add=True)

    def body(i, carry):
        j = 2 * i
        for b in range(2):
            pltpu.make_async_copy(
                ones_v, acc_sh.at[sidx_v.at[j + b]], ssems[b]).wait()
            pltpu.async_copy(
                ones_v, acc_sh.at[sidx_v.at[j + b + 2]], ssems[b], add=True)
        return carry

    lax.fori_loop(0, NCHW // 2, body, 0)
    for b in range(2):
        pltpu.make_async_copy(
            ones_v, acc_sh.at[sidx_v.at[NCHW + b]], ssems[b]).wait()
    plsc.subcore_barrier()
    pltpu.sync_copy(acc_sh.at[pl.ds(s * ZR, ZR)], out_hbm.at[c].at[pl.ds(s * ZR, ZR)])


@functools.partial(
    pl.kernel,
    out_type=jax.ShapeDtypeStruct((2, ROWS, 64), jnp.float32),
    mesh=_mesh,
    compiler_params=_sc_params,
    scratch_types=[
        pltpu.VMEM((16, CH), jnp.int32),
        pltpu.VMEM((16, CH), jnp.int32),
        pltpu.VMEM((16, CH), jnp.int32),
        pltpu.VMEM((16, CH), jnp.int32),
        pltpu.VMEM((CH, 64), jnp.float32),
        pltpu.VMEM((CH, 64), jnp.float32),
        pltpu.VMEM_SHARED((ROWS, 64), jnp.float32),
        pltpu.SemaphoreType.DMA,
        pltpu.SemaphoreType.DMA,
    ],
)
def _sc_layer_agg(g_hbm, gidx_hbm, sidx_hbm, zeros_hbm, out_hbm,
                  ga, sa, gb, sb, r0, r1, acc_sh, gs0, gs1):
    gsl = (ga, gb)
    ssl = (sa, sb)
    rows = (r0, r1)
    gsems = (gs0, gs1)
    c = lax.axis_index("c")
    s = lax.axis_index("s")
    base = s * NCHT
    pltpu.sync_copy(zeros_hbm.at[pl.ds(s * ZR, ZR)], acc_sh.at[pl.ds(s * ZR, ZR)])
    plsc.subcore_barrier()

    pltpu.sync_copy(gidx_hbm.at[pl.ds(base, 16)], ga)
    pltpu.sync_copy(sidx_hbm.at[pl.ds(base, 16)], sa)
    pltpu.async_copy(g_hbm.at[c].at[ga.at[0]], rows[0], gsems[0])

    def pair(p, carry):
        for s2 in (0, 1):
            k = 2 * p + s2
            cur_g, cur_s = gsl[s2], ssl[s2]
            nxt_g, nxt_s = gsl[1 - s2], ssl[1 - s2]
            off = base + jnp.minimum((k + 1) * 16, NCHT - 16)
            pltpu.sync_copy(gidx_hbm.at[pl.ds(off, 16)], nxt_g)
            pltpu.sync_copy(sidx_hbm.at[pl.ds(off, 16)], nxt_s)
            for u in range(16):
                b = u % 2
                pltpu.make_async_copy(
                    g_hbm.at[c].at[cur_g.at[u]], rows[b], gsems[b]).wait()
                if u < 15:
                    pltpu.async_copy(
                        g_hbm.at[c].at[cur_g.at[u + 1]], rows[1 - b],
                        gsems[1 - b])
                else:
                    pltpu.async_copy(
                        g_hbm.at[c].at[nxt_g.at[0]], rows[1 - b], gsems[1 - b])
                pltpu.sync_copy(rows[b], acc_sh.at[cur_s.at[u]], add=True)
        return carry

    lax.fori_loop(0, NCHT // 32, pair, 0)
    pltpu.make_async_copy(
        g_hbm.at[c].at[ga.at[0]], rows[0], gsems[0]).wait()
    plsc.subcore_barrier()
    pltpu.sync_copy(acc_sh.at[pl.ds(s * ZR, ZR)], out_hbm.at[c].at[pl.ds(s * ZR, ZR)])


@functools.partial(
    pl.kernel,
    out_type=jax.ShapeDtypeStruct((2, NPAD, 16), jnp.float32),
    mesh=_mesh,
    compiler_params=_sc_params,
    scratch_types=[
        pltpu.VMEM((NCHW, CHL), jnp.int32),
        pltpu.VMEM((NCHW, CHL), jnp.int32),
        pltpu.VMEM((CHL, 16), jnp.float32),
        pltpu.VMEM((CHL, 16), jnp.float32),
        pltpu.VMEM_SHARED((NPAD, 16), jnp.float32),
        pltpu.SemaphoreType.DMA,
        pltpu.SemaphoreType.DMA,
        pltpu.SemaphoreType.DMA,
        pltpu.SemaphoreType.DMA,
    ],
)
def _sc_last_agg(g_hbm, gidx_hbm, sidx_hbm, zeros_hbm, out_hbm,
                 gidx_v, sidx_v, r0, r1, acc_sh,
                 gs0, gs1, ss0, ss1):
    rows = (r0, r1)
    gsems = (gs0, gs1)
    ssems = (ss0, ss1)
    c = lax.axis_index("c")
    s = lax.axis_index("s")
    base = (c * 16 + s) * NCHW
    pltpu.sync_copy(gidx_hbm.at[pl.ds(base, NCHW)], gidx_v)
    pltpu.sync_copy(sidx_hbm.at[pl.ds(base, NCHW)], sidx_v)
    pltpu.sync_copy(zeros_hbm.at[pl.ds(s * ZR2, ZR2)], acc_sh.at[pl.ds(s * ZR2, ZR2)])
    plsc.subcore_barrier()

    pltpu.async_copy(g_hbm.at[gidx_v.at[0]], rows[0], gsems[0])
    pltpu.make_async_copy(g_hbm.at[gidx_v.at[0]], rows[0], gsems[0]).wait()
    pltpu.async_copy(rows[0], acc_sh.at[sidx_v.at[0]], ssems[0], add=True)
    pltpu.async_copy(g_hbm.at[gidx_v.at[1]], rows[1], gsems[1])

    def ring(i, carry):
        j0 = 2 * i + 1
        for b in (1, 0):
            j = j0 + (1 - b)
            pltpu.make_async_copy(
                g_hbm.at[gidx_v.at[j]], rows[b], gsems[b]).wait()
            pltpu.make_async_copy(
                rows[1 - b], acc_sh.at[sidx_v.at[j - 1]], ssems[1 - b]).wait()
            jn = jnp.minimum(j + 1, NCHW - 1)
            pltpu.async_copy(g_hbm.at[gidx_v.at[jn]], rows[1 - b], gsems[1 - b])
            pltpu.async_copy(rows[b], acc_sh.at[sidx_v.at[j]], ssems[b], add=True)
        return carry

    lax.fori_loop(0, (NCHW - 2) // 2, ring, 0)
    pltpu.make_async_copy(g_hbm.at[gidx_v.at[NCHW - 1]], rows[1], gsems[1]).wait()
    pltpu.async_copy(rows[1], acc_sh.at[sidx_v.at[NCHW - 1]], ssems[1], add=True)
    pltpu.make_async_copy(
        rows[0], acc_sh.at[sidx_v.at[NCHW - 2]], ssems[0]).wait()
    pltpu.make_async_copy(
        rows[1], acc_sh.at[sidx_v.at[NCHW - 1]], ssems[1]).wait()
    plsc.subcore_barrier()
    pltpu.sync_copy(acc_sh.at[pl.ds(s * ZR2, ZR2)], out_hbm.at[c].at[pl.ds(s * ZR2, ZR2)])


def _prep_idx_body(src_ref, dst_ref, rev_ref, sidx_ref, gidx_ref):
    rev = rev_ref[...]
    sidx_ref[...] = dst_ref[...] + NPAD * rev
    gidx_ref[...] = src_ref[...] + NPAD * rev


def _tc_prep_idx(srcp, dstp, revp):
    blk = pl.BlockSpec((32, CH), lambda r: (r, 0))
    return pl.pallas_call(
        _prep_idx_body,
        grid=(ER // 32,),
        in_specs=[blk, blk, blk],
        out_specs=[blk, blk],
        out_shape=[jax.ShapeDtypeStruct((ER, CH), jnp.int32)] * 2,
    )(srcp, dstp, revp)


def _deg_fin_body(dac_ref, dis_ref, disall_ref):
    dac = dac_ref[...]
    cnt_st = dac[0, 0, :, 0:1] + dac[1, 0, :, 0:1]
    cnt_ts = dac[0, 1, :, 0:1] + dac[1, 1, :, 0:1]
    dis_ref[0] = lax.rsqrt(cnt_st + 1.0)
    dis_ref[1] = lax.rsqrt(cnt_ts + 1.0)
    disall_ref[...] = lax.rsqrt(cnt_st + cnt_ts + 1.0)


def _tc_deg_fin(degacc4):
    return pl.pallas_call(
        _deg_fin_body,
        grid=(NPAD // 256,),
        in_specs=[pl.BlockSpec((2, 2, 256, 16), lambda r: (0, 0, r, 0))],
        out_specs=[pl.BlockSpec((2, 256, 1), lambda r: (0, r, 0)),
                   pl.BlockSpec((256, 1), lambda r: (r, 0))],
        out_shape=[jax.ShapeDtypeStruct((2, NPAD, 1), jnp.float32),
                   jax.ShapeDtypeStruct((NPAD, 1), jnp.float32)],
    )(degacc4)


def _fwd_body(h_ref, w_ref, dis_ref, out_ref):
    hw = jnp.dot(h_ref[...], w_ref[0], preferred_element_type=jnp.float32)
    out_ref[0, 0] = dis_ref[0] * hw


def _tc_fwd(h, wcat, dis):
    fin = h.shape[1]
    wq = jnp.transpose(wcat.reshape(fin, 4, 64), (1, 0, 2))
    return pl.pallas_call(
        _fwd_body,
        grid=(NPAD // 256, 2, 2),
        in_specs=[pl.BlockSpec((256, fin), lambda r, p, c: (r, 0)),
                  pl.BlockSpec((1, fin, 64), lambda r, p, c: (2 * p + c, 0, 0)),
                  pl.BlockSpec((1, 256, 1), lambda r, p, c: (p, r, 0))],
        out_specs=pl.BlockSpec((1, 1, 256, 64), lambda r, p, c: (c, p, r, 0)),
        out_shape=jax.ShapeDtypeStruct((2, 2, NPAD, 64), jnp.float32),
    )(h, wq, dis)


def _epi_body(acc_ref, g_ref, dis_ref, bq_ref, h_ref):
    a = acc_ref[...]
    g = g_ref[...]
    d = dis_ref[...]
    cols = []
    for p in range(2):
        for cc in range(2):
            cols.append(jnp.maximum(d[p] * (a[cc, p] + g[cc, p]) + bq_ref[cc, p], 0.0))
    h_ref[...] = jnp.concatenate(cols, axis=1)


def _tc_epi(acc4, g4, dis, bq):
    return pl.pallas_call(
        _epi_body,
        grid=(NPAD // 256,),
        in_specs=[pl.BlockSpec((2, 2, 256, 64), lambda r: (0, 0, r, 0)),
                  pl.BlockSpec((2, 2, 256, 64), lambda r: (0, 0, r, 0)),
                  pl.BlockSpec((2, 256, 1), lambda r: (0, r, 0)),
                  pl.BlockSpec((2, 2, 1, 64), lambda r: (0, 0, 0, 0))],
        out_specs=pl.BlockSpec((256, 256), lambda r: (r, 0)),
        out_shape=jax.ShapeDtypeStruct((NPAD, 256), jnp.float32),
    )(acc4, g4, dis, bq)


def _lastmm_body(h_ref, w_ref, dis_ref, g_ref):
    hw = jnp.dot(h_ref[...], w_ref[...], preferred_element_type=jnp.float32)
    g_ref[...] = dis_ref[...] * hw


def _tc_lastmm(h, w_last, dis_all):
    return pl.pallas_call(
        _lastmm_body,
        grid=(NPAD // 256,),
        in_specs=[pl.BlockSpec((256, 256), lambda r: (r, 0)),
                  pl.BlockSpec((256, 16), lambda r: (0, 0)),
                  pl.BlockSpec((256, 1), lambda r: (r, 0))],
        out_specs=pl.BlockSpec((256, 16), lambda r: (r, 0)),
        out_shape=jax.ShapeDtypeStruct((NPAD, 16), jnp.float32),
    )(h, w_last, dis_all)


def _final_body(acc_ref, g_ref, dis_ref, b_ref, out_ref):
    o = dis_ref[...] * (acc_ref[0] + acc_ref[1] + g_ref[...]) + b_ref[...]
    m = jnp.max(o, axis=1, keepdims=True)
    e = o - m
    out_ref[...] = e - jnp.log(jnp.sum(jnp.exp(e), axis=1, keepdims=True))


def _tc_final(acc2, g2, dis_all, b_last):
    return pl.pallas_call(
        _final_body,
        grid=(NPAD // 256,),
        in_specs=[pl.BlockSpec((2, 256, 16), lambda r: (0, r, 0)),
                  pl.BlockSpec((256, 16), lambda r: (r, 0)),
                  pl.BlockSpec((256, 1), lambda r: (r, 0)),
                  pl.BlockSpec((1, 16), lambda r: (0, 0))],
        out_specs=pl.BlockSpec((256, 16), lambda r: (r, 0)),
        out_shape=jax.ShapeDtypeStruct((NPAD, 16), jnp.float32),
    )(acc2, g2, dis_all, b_last)


def kernel(x, edge_index, is_reversed, W_st0, b_st0, W_ts0, b_ts0,
           W_st1, b_st1, W_ts1, b_ts1, W_last, b_last):
    src = edge_index[0]
    dst = edge_index[1]
    rev = is_reversed.astype(jnp.int32)
    pad = EPAD - E
    srcp = jnp.pad(src, (0, pad)).reshape(ER, CH)
    dstp = jnp.pad(dst, (0, pad), constant_values=N).reshape(ER, CH)
    revp = jnp.pad(rev, (0, pad)).reshape(ER, CH)

    sidx, gidx = _tc_prep_idx(srcp, dstp, revp)
    sidx64 = sidx.reshape(ERL, CHL)

    zeros16 = jnp.zeros((ROWS, 16), jnp.float32)
    zeros64 = jnp.zeros((ROWS, 64), jnp.float32)
    ones16 = jnp.ones((CHL, 16), jnp.float32)

    degacc = _sc_degree(sidx64, ones16, zeros16)
    dis, dis_all = _tc_deg_fin(degacc.reshape(2, 2, NPAD, 16))

    xpad = jnp.pad(x, ((0, NPAD - N), (0, 0)))
    wcat0 = jnp.concatenate([W_st0, W_ts0], axis=1)
    wcat1 = jnp.concatenate([W_st1, W_ts1], axis=1)

    def bias_quads(b_st, b_ts):
        return jnp.stack([
            jnp.stack([b_st[0:64], b_ts[0:64]]),
            jnp.stack([b_st[64:128], b_ts[64:128]]),
        ])[:, :, None, :]

    bq0 = bias_quads(b_st0, b_ts0)
    bq1 = bias_quads(b_st1, b_ts1)

    g0 = _tc_fwd(xpad, wcat0, dis)
    acc0 = _sc_layer_agg(g0.reshape(2, ROWS, 64), gidx, sidx, zeros64)
    h1 = _tc_epi(acc0.reshape(2, 2, NPAD, 64), g0, dis, bq0)

    g1 = _tc_fwd(h1, wcat1, dis)
    acc1 = _sc_layer_agg(g1.reshape(2, ROWS, 64), gidx, sidx, zeros64)
    h2 = _tc_epi(acc1.reshape(2, 2, NPAD, 64), g1, dis, bq1)

    g2 = _tc_lastmm(h2, W_last, dis_all)
    acc2 = _sc_last_agg(g2, srcp.reshape(ERL, CHL), dstp.reshape(ERL, CHL), zeros16)
    out = _tc_final(acc2, g2, dis_all, b_last[None, :])
    return out[:N]


# speedup vs baseline: 5.9631x; 1.1505x over previous
import functools

import jax
import jax.numpy as jnp
from jax import lax
from jax.experimental import pallas as pl
from jax.experimental.pallas import tpu as pltpu
from jax.experimental.pallas import tpu_sc as plsc

N = 10000
NPAD = 10240
E = 320000
CH = 128
ER = 2560
EPAD = ER * CH
CHL = 64
ERL = EPAD // CHL
NCHT = ER // 16
NCHW = ERL // 32
ROWS = 2 * NPAD
ZR = ROWS // 16
ZR2 = NPAD // 16

_mesh = plsc.VectorSubcoreMesh(core_axis_name="c", subcore_axis_name="s")
_sc_params = pltpu.CompilerParams(use_tc_tiling_on_sc=False)


@functools.partial(
    pl.kernel,
    out_type=jax.ShapeDtypeStruct((2, ROWS, 16), jnp.float32),
    mesh=_mesh,
    compiler_params=_sc_params,
    scratch_types=[
        pltpu.VMEM((NCHW + 2, CHL), jnp.int32),
        pltpu.VMEM((CHL, 16), jnp.float32),
        pltpu.VMEM_SHARED((ROWS, 16), jnp.float32),
        pltpu.SemaphoreType.DMA,
        pltpu.SemaphoreType.DMA,
    ],
)
def _sc_degree(sidx_hbm, ones_hbm, zeros_hbm, out_hbm, sidx_v, ones_v, acc_sh,
               ss0, ss1):
    ssems = (ss0, ss1)
    c = lax.axis_index("c")
    s = lax.axis_index("s")
    base = (c * 16 + s) * NCHW
    pltpu.sync_copy(sidx_hbm.at[pl.ds(base, NCHW)], sidx_v.at[pl.ds(0, NCHW)])
    trash = jnp.full((16,), N, jnp.int32)
    for k in range(CHL // 16):
        sidx_v[NCHW, pl.ds(k * 16, 16)] = trash
        sidx_v[NCHW + 1, pl.ds(k * 16, 16)] = trash
    pltpu.sync_copy(ones_hbm, ones_v)
    pltpu.sync_copy(zeros_hbm.at[pl.ds(s * ZR, ZR)], acc_sh.at[pl.ds(s * ZR, ZR)])
    plsc.subcore_barrier()

    pltpu.async_copy(ones_v, acc_sh.at[sidx_v.at[0]], ssems[0], add=True)
    pltpu.async_copy(ones_v, acc_sh.at[sidx_v.at[1]], ssems[1], add=True)

    def body(i, carry):
        j = 2 * i
        for b in range(2):
            pltpu.make_async_copy(
                ones_v, acc_sh.at[sidx_v.at[j + b]], ssems[b]).wait()
            pltpu.async_copy(
                ones_v, acc_sh.at[sidx_v.at[j + b + 2]], ssems[b], add=True)
        return carry

    lax.fori_loop(0, NCHW // 2, body, 0)
    for b in range(2):
        pltpu.make_async_copy(
            ones_v, acc_sh.at[sidx_v.at[NCHW + b]], ssems[b]).wait()
    plsc.subcore_barrier()
    pltpu.sync_copy(acc_sh.at[pl.ds(s * ZR, ZR)], out_hbm.at[c].at[pl.ds(s * ZR, ZR)])


@functools.partial(
    pl.kernel,
    out_type=jax.ShapeDtypeStruct((2, ROWS, 64), jnp.float32),
    mesh=_mesh,
    compiler_params=_sc_params,
    scratch_types=[
        pltpu.VMEM((16, CH), jnp.int32),
        pltpu.VMEM((16, CH), jnp.int32),
        pltpu.VMEM((16, CH), jnp.int32),
        pltpu.VMEM((16, CH), jnp.int32),
        pltpu.VMEM((CH, 64), jnp.float32),
        pltpu.VMEM((CH, 64), jnp.float32),
        pltpu.VMEM((CH, 64), jnp.float32),
        pltpu.VMEM((CH, 64), jnp.float32),
        pltpu.VMEM_SHARED((ROWS, 64), jnp.float32),
        pltpu.SemaphoreType.DMA,
        pltpu.SemaphoreType.DMA,
        pltpu.SemaphoreType.DMA,
        pltpu.SemaphoreType.DMA,
    ],
)
def _sc_layer_agg(g_hbm, gidx_hbm, sidx_hbm, zeros_hbm, out_hbm,
                  ga, sa, gb, sb, r0, r1, r2, r3, acc_sh, gs0, gs1, gs2, gs3):
    gsl = (ga, gb)
    ssl = (sa, sb)
    rows = (r0, r1, r2, r3)
    gsems = (gs0, gs1, gs2, gs3)
    c = lax.axis_index("c")
    s = lax.axis_index("s")
    base = s * NCHT
    pltpu.sync_copy(zeros_hbm.at[pl.ds(s * ZR, ZR)], acc_sh.at[pl.ds(s * ZR, ZR)])
    plsc.subcore_barrier()

    pltpu.sync_copy(gidx_hbm.at[pl.ds(base, 16)], ga)
    pltpu.sync_copy(sidx_hbm.at[pl.ds(base, 16)], sa)
    pltpu.async_copy(g_hbm.at[c].at[ga.at[0]], rows[0], gsems[0])
    pltpu.async_copy(g_hbm.at[c].at[ga.at[1]], rows[1], gsems[1])

    def pair(p, carry):
        for s2 in (0, 1):
            k = 2 * p + s2
            cur_g, cur_s = gsl[s2], ssl[s2]
            nxt_g, nxt_s = gsl[1 - s2], ssl[1 - s2]
            off = base + jnp.minimum((k + 1) * 16, NCHT - 16)
            pltpu.sync_copy(gidx_hbm.at[pl.ds(off, 16)], nxt_g)
            pltpu.sync_copy(sidx_hbm.at[pl.ds(off, 16)], nxt_s)
            for u in range(16):
                b = u % 4
                bn = (u + 2) % 4
                pltpu.make_async_copy(
                    g_hbm.at[c].at[cur_g.at[u]], rows[b], gsems[b]).wait()
                if u < 14:
                    pltpu.async_copy(
                        g_hbm.at[c].at[cur_g.at[u + 2]], rows[bn], gsems[bn])
                else:
                    pltpu.async_copy(
                        g_hbm.at[c].at[nxt_g.at[u - 14]], rows[bn], gsems[bn])
                pltpu.sync_copy(rows[b], acc_sh.at[cur_s.at[u]], add=True)
        return carry

    lax.fori_loop(0, NCHT // 32, pair, 0)
    pltpu.make_async_copy(
        g_hbm.at[c].at[ga.at[0]], rows[0], gsems[0]).wait()
    pltpu.make_async_copy(
        g_hbm.at[c].at[ga.at[1]], rows[1], gsems[1]).wait()
    plsc.subcore_barrier()
    pltpu.sync_copy(acc_sh.at[pl.ds(s * ZR, ZR)], out_hbm.at[c].at[pl.ds(s * ZR, ZR)])


@functools.partial(
    pl.kernel,
    out_type=jax.ShapeDtypeStruct((2, NPAD, 16), jnp.float32),
    mesh=_mesh,
    compiler_params=_sc_params,
    scratch_types=[
        pltpu.VMEM((NCHW, CHL), jnp.int32),
        pltpu.VMEM((NCHW, CHL), jnp.int32),
        pltpu.VMEM((CHL, 16), jnp.float32),
        pltpu.VMEM((CHL, 16), jnp.float32),
        pltpu.VMEM_SHARED((NPAD, 16), jnp.float32),
        pltpu.SemaphoreType.DMA,
        pltpu.SemaphoreType.DMA,
        pltpu.SemaphoreType.DMA,
        pltpu.SemaphoreType.DMA,
    ],
)
def _sc_last_agg(g_hbm, gidx_hbm, sidx_hbm, zeros_hbm, out_hbm,
                 gidx_v, sidx_v, r0, r1, acc_sh,
                 gs0, gs1, ss0, ss1):
    rows = (r0, r1)
    gsems = (gs0, gs1)
    ssems = (ss0, ss1)
    c = lax.axis_index("c")
    s = lax.axis_index("s")
    base = (c * 16 + s) * NCHW
    pltpu.sync_copy(gidx_hbm.at[pl.ds(base, NCHW)], gidx_v)
    pltpu.sync_copy(sidx_hbm.at[pl.ds(base, NCHW)], sidx_v)
    pltpu.sync_copy(zeros_hbm.at[pl.ds(s * ZR2, ZR2)], acc_sh.at[pl.ds(s * ZR2, ZR2)])
    plsc.subcore_barrier()

    pltpu.async_copy(g_hbm.at[gidx_v.at[0]], rows[0], gsems[0])
    pltpu.make_async_copy(g_hbm.at[gidx_v.at[0]], rows[0], gsems[0]).wait()
    pltpu.async_copy(rows[0], acc_sh.at[sidx_v.at[0]], ssems[0], add=True)
    pltpu.async_copy(g_hbm.at[gidx_v.at[1]], rows[1], gsems[1])

    def ring(i, carry):
        j0 = 2 * i + 1
        for b in (1, 0):
            j = j0 + (1 - b)
            pltpu.make_async_copy(
                g_hbm.at[gidx_v.at[j]], rows[b], gsems[b]).wait()
            pltpu.make_async_copy(
                rows[1 - b], acc_sh.at[sidx_v.at[j - 1]], ssems[1 - b]).wait()
            jn = jnp.minimum(j + 1, NCHW - 1)
            pltpu.async_copy(g_hbm.at[gidx_v.at[jn]], rows[1 - b], gsems[1 - b])
            pltpu.async_copy(rows[b], acc_sh.at[sidx_v.at[j]], ssems[b], add=True)
        return carry

    lax.fori_loop(0, (NCHW - 2) // 2, ring, 0)
    pltpu.make_async_copy(g_hbm.at[gidx_v.at[NCHW - 1]], rows[1], gsems[1]).wait()
    pltpu.async_copy(rows[1], acc_sh.at[sidx_v.at[NCHW - 1]], ssems[1], add=True)
    pltpu.make_async_copy(
        rows[0], acc_sh.at[sidx_v.at[NCHW - 2]], ssems[0]).wait()
    pltpu.make_async_copy(
        rows[1], acc_sh.at[sidx_v.at[NCHW - 1]], ssems[1]).wait()
    plsc.subcore_barrier()
    pltpu.sync_copy(acc_sh.at[pl.ds(s * ZR2, ZR2)], out_hbm.at[c].at[pl.ds(s * ZR2, ZR2)])


def _prep_idx_body(src_ref, dst_ref, rev_ref, sidx_ref, gidx_ref):
    rev = rev_ref[...]
    sidx_ref[...] = dst_ref[...] + NPAD * rev
    gidx_ref[...] = src_ref[...] + NPAD * rev


def _tc_prep_idx(srcp, dstp, revp):
    blk = pl.BlockSpec((32, CH), lambda r: (r, 0))
    return pl.pallas_call(
        _prep_idx_body,
        grid=(ER // 32,),
        in_specs=[blk, blk, blk],
        out_specs=[blk, blk],
        out_shape=[jax.ShapeDtypeStruct((ER, CH), jnp.int32)] * 2,
    )(srcp, dstp, revp)


def _deg_fin_body(dac_ref, dis_ref, disall_ref):
    dac = dac_ref[...]
    cnt_st = dac[0, 0, :, 0:1] + dac[1, 0, :, 0:1]
    cnt_ts = dac[0, 1, :, 0:1] + dac[1, 1, :, 0:1]
    dis_ref[0] = lax.rsqrt(cnt_st + 1.0)
    dis_ref[1] = lax.rsqrt(cnt_ts + 1.0)
    disall_ref[...] = lax.rsqrt(cnt_st + cnt_ts + 1.0)


def _tc_deg_fin(degacc4):
    return pl.pallas_call(
        _deg_fin_body,
        grid=(NPAD // 256,),
        in_specs=[pl.BlockSpec((2, 2, 256, 16), lambda r: (0, 0, r, 0))],
        out_specs=[pl.BlockSpec((2, 256, 1), lambda r: (0, r, 0)),
                   pl.BlockSpec((256, 1), lambda r: (r, 0))],
        out_shape=[jax.ShapeDtypeStruct((2, NPAD, 1), jnp.float32),
                   jax.ShapeDtypeStruct((NPAD, 1), jnp.float32)],
    )(degacc4)


def _fwd_body(h_ref, w_ref, dis_ref, out_ref):
    hw = jnp.dot(h_ref[...], w_ref[0], preferred_element_type=jnp.float32)
    out_ref[0, 0] = dis_ref[0] * hw


def _tc_fwd(h, wcat, dis):
    fin = h.shape[1]
    wq = jnp.transpose(wcat.reshape(fin, 4, 64), (1, 0, 2))
    return pl.pallas_call(
        _fwd_body,
        grid=(NPAD // 256, 2, 2),
        in_specs=[pl.BlockSpec((256, fin), lambda r, p, c: (r, 0)),
                  pl.BlockSpec((1, fin, 64), lambda r, p, c: (2 * p + c, 0, 0)),
                  pl.BlockSpec((1, 256, 1), lambda r, p, c: (p, r, 0))],
        out_specs=pl.BlockSpec((1, 1, 256, 64), lambda r, p, c: (c, p, r, 0)),
        out_shape=jax.ShapeDtypeStruct((2, 2, NPAD, 64), jnp.float32),
    )(h, wq, dis)


def _epi_body(acc_ref, g_ref, dis_ref, bq_ref, h_ref):
    a = acc_ref[...]
    g = g_ref[...]
    d = dis_ref[...]
    cols = []
    for p in range(2):
        for cc in range(2):
            cols.append(jnp.maximum(d[p] * (a[cc, p] + g[cc, p]) + bq_ref[cc, p], 0.0))
    h_ref[...] = jnp.concatenate(cols, axis=1)


def _tc_epi(acc4, g4, dis, bq):
    return pl.pallas_call(
        _epi_body,
        grid=(NPAD // 256,),
        in_specs=[pl.BlockSpec((2, 2, 256, 64), lambda r: (0, 0, r, 0)),
                  pl.BlockSpec((2, 2, 256, 64), lambda r: (0, 0, r, 0)),
                  pl.BlockSpec((2, 256, 1), lambda r: (0, r, 0)),
                  pl.BlockSpec((2, 2, 1, 64), lambda r: (0, 0, 0, 0))],
        out_specs=pl.BlockSpec((256, 256), lambda r: (r, 0)),
        out_shape=jax.ShapeDtypeStruct((NPAD, 256), jnp.float32),
    )(acc4, g4, dis, bq)


def _lastmm_body(h_ref, w_ref, dis_ref, g_ref):
    hw = jnp.dot(h_ref[...], w_ref[...], preferred_element_type=jnp.float32)
    g_ref[...] = dis_ref[...] * hw


def _tc_lastmm(h, w_last, dis_all):
    return pl.pallas_call(
        _lastmm_body,
        grid=(NPAD // 256,),
        in_specs=[pl.BlockSpec((256, 256), lambda r: (r, 0)),
                  pl.BlockSpec((256, 16), lambda r: (0, 0)),
                  pl.BlockSpec((256, 1), lambda r: (r, 0))],
        out_specs=pl.BlockSpec((256, 16), lambda r: (r, 0)),
        out_shape=jax.ShapeDtypeStruct((NPAD, 16), jnp.float32),
    )(h, w_last, dis_all)


def _final_body(acc_ref, g_ref, dis_ref, b_ref, out_ref):
    o = dis_ref[...] * (acc_ref[0] + acc_ref[1] + g_ref[...]) + b_ref[...]
    m = jnp.max(o, axis=1, keepdims=True)
    e = o - m
    out_ref[...] = e - jnp.log(jnp.sum(jnp.exp(e), axis=1, keepdims=True))


def _tc_final(acc2, g2, dis_all, b_last):
    return pl.pallas_call(
        _final_body,
        grid=(NPAD // 256,),
        in_specs=[pl.BlockSpec((2, 256, 16), lambda r: (0, r, 0)),
                  pl.BlockSpec((256, 16), lambda r: (r, 0)),
                  pl.BlockSpec((256, 1), lambda r: (r, 0)),
                  pl.BlockSpec((1, 16), lambda r: (0, 0))],
        out_specs=pl.BlockSpec((256, 16), lambda r: (r, 0)),
        out_shape=jax.ShapeDtypeStruct((NPAD, 16), jnp.float32),
    )(acc2, g2, dis_all, b_last)


def kernel(x, edge_index, is_reversed, W_st0, b_st0, W_ts0, b_ts0,
           W_st1, b_st1, W_ts1, b_ts1, W_last, b_last):
    src = edge_index[0]
    dst = edge_index[1]
    rev = is_reversed.astype(jnp.int32)
    pad = EPAD - E
    srcp = jnp.pad(src, (0, pad)).reshape(ER, CH)
    dstp = jnp.pad(dst, (0, pad), constant_values=N).reshape(ER, CH)
    revp = jnp.pad(rev, (0, pad)).reshape(ER, CH)

    sidx, gidx = _tc_prep_idx(srcp, dstp, revp)
    sidx64 = sidx.reshape(ERL, CHL)

    zeros16 = jnp.zeros((ROWS, 16), jnp.float32)
    zeros64 = jnp.zeros((ROWS, 64), jnp.float32)
    ones16 = jnp.ones((CHL, 16), jnp.float32)

    degacc = _sc_degree(sidx64, ones16, zeros16)
    dis, dis_all = _tc_deg_fin(degacc.reshape(2, 2, NPAD, 16))

    xpad = jnp.pad(x, ((0, NPAD - N), (0, 0)))
    wcat0 = jnp.concatenate([W_st0, W_ts0], axis=1)
    wcat1 = jnp.concatenate([W_st1, W_ts1], axis=1)

    def bias_quads(b_st, b_ts):
        return jnp.stack([
            jnp.stack([b_st[0:64], b_ts[0:64]]),
            jnp.stack([b_st[64:128], b_ts[64:128]]),
        ])[:, :, None, :]

    bq0 = bias_quads(b_st0, b_ts0)
    bq1 = bias_quads(b_st1, b_ts1)

    g0 = _tc_fwd(xpad, wcat0, dis)
    acc0 = _sc_layer_agg(g0.reshape(2, ROWS, 64), gidx, sidx, zeros64)
    h1 = _tc_epi(acc0.reshape(2, 2, NPAD, 64), g0, dis, bq0)

    g1 = _tc_fwd(h1, wcat1, dis)
    acc1 = _sc_layer_agg(g1.reshape(2, ROWS, 64), gidx, sidx, zeros64)
    h2 = _tc_epi(acc1.reshape(2, 2, NPAD, 64), g1, dis, bq1)

    g2 = _tc_lastmm(h2, W_last, dis_all)
    acc2 = _sc_last_agg(g2, srcp.reshape(ERL, CHL), dstp.reshape(ERL, CHL), zeros16)
    out = _tc_final(acc2, g2, dis_all, b_last[None, :])
    return out[:N]
